# baseline (device time: 51298 ns/iter reference)
import jax
import jax.numpy as jnp
from jax import lax
from jax.experimental import pallas as pl
from jax.experimental.pallas import tpu as pltpu

QR = 1024
NSUB = 16
HALF = NSUB // 2
SUB = QR // NSUB

ZRAW = 0 * NSUB
XQ = 1 * NSUB
YQ = 2 * NSUB
FWDX = 3 * NSUB
FWDY = 3 * NSUB + HALF
NSEM = 4 * NSUB

O_ME, O_X, O_Y, O_D = 0, NSUB, 2 * NSUB, 3 * NSUB


def kernel(x):
    m, n = x.shape

    def body(
        x_hbm, out_hbm, lraw, sraw, praw, red, qx, qy, qd,
        local_sem, out_sems, send_sems, recv_sems,
    ):
        mx = lax.axis_index("x")
        my = lax.axis_index("y")
        mz = lax.axis_index("z")
        nbr_x = (1 - mx, my, mz)
        nbr_y = (mx, 1 - my, mz)
        nbr_z = (mx, my, 1 - mz)

        q_me = 2 * mx + my
        q_x = 2 * (1 - mx) + my
        q_y = 2 * mx + (1 - my)
        q_d = 2 * (1 - mx) + (1 - my)

        def sub(ref, s):
            return ref.at[pl.ds(s * SUB, SUB), :]

        def copy(slot, src, dst, target):
            return pltpu.make_async_remote_copy(
                src_ref=src,
                dst_ref=dst,
                send_sem=send_sems.at[slot],
                recv_sem=recv_sems.at[slot],
                device_id=target,
                device_id_type=pl.DeviceIdType.MESH,
            )

        out_dmas = []

        def out_sub(slot, src_ref, s, quarter):
            d = pltpu.make_async_copy(
                sub(src_ref, s),
                out_hbm.at[pl.ds(quarter * QR + s * SUB, SUB), :],
                out_sems.at[slot + s],
            )
            d.start()
            out_dmas.append(d)

        ldma = pltpu.make_async_copy(
            x_hbm.at[pl.ds(q_me * QR, QR), :], lraw, local_sem
        )
        ldma.start()
        ldma.wait()
        sraw[pl.ds(0, SUB), :] = lraw[pl.ds(0, SUB), :].astype(jnp.bfloat16)

        barrier_sem = pltpu.get_barrier_semaphore()
        for nbr in (nbr_x, nbr_y, nbr_z):
            pl.semaphore_signal(
                barrier_sem, inc=1,
                device_id=nbr, device_id_type=pl.DeviceIdType.MESH,
            )
        pl.semaphore_wait(barrier_sem, 3)

        sends = []

        rz = []
        for s in range(NSUB):
            if s:
                sraw[pl.ds(s * SUB, SUB), :] = lraw[
                    pl.ds(s * SUB, SUB), :
                ].astype(jnp.bfloat16)
            r = copy(ZRAW + s, sub(sraw, s), sub(praw, s), nbr_z)
            r.start()
            rz.append(r)
            sends.append(r)

        rqx, rqy = {}, {}
        for s in range(NSUB):
            rz[s].wait_recv()
            red_f32 = (
                lraw[pl.ds(s * SUB, SUB), :]
                + praw[pl.ds(s * SUB, SUB), :].astype(jnp.float32)
            )
            red[pl.ds(s * SUB, SUB), :] = red_f32.astype(jnp.bfloat16)
            rx = copy(XQ + s, sub(red, s), sub(qx, s), nbr_x)
            ry = copy(YQ + s, sub(red, s), sub(qy, s), nbr_y)
            rx.start()
            ry.start()
            rqx[s], rqy[s] = rx, ry
            sends.extend((rx, ry))
            out_sub(O_ME, red, s, q_me)

        for s in range(HALF):
            rqy[s].wait_recv()
            f = copy(FWDX + s, sub(qy, s), sub(qd, s), nbr_x)
            f.start()
            sends.append(f)
            out_sub(O_Y, qy, s, q_y)
        for s in range(HALF, NSUB):
            rqx[s].wait_recv()
            f = copy(FWDY + (s - HALF), sub(qx, s), sub(qd, s), nbr_y)
            f.start()
            sends.append(f)
            out_sub(O_X, qx, s, q_x)

        for s in range(HALF):
            rqx[s].wait_recv()
            out_sub(O_X, qx, s, q_x)
        for s in range(HALF, NSUB):
            rqy[s].wait_recv()
            out_sub(O_Y, qy, s, q_y)

        for s in range(NSUB):
            slot = FWDX + s if s < HALF else FWDY + (s - HALF)
            src_nbr = nbr_x if s < HALF else nbr_y
            copy(slot, sub(qy, s), sub(qd, s), src_nbr).wait_recv()
            out_sub(O_D, qd, s, q_d)

        for d in sends:
            d.wait_send()
        for d in out_dmas:
            d.wait()

    return pl.pallas_call(
        body,
        out_shape=jax.ShapeDtypeStruct((m, n), jnp.bfloat16),
        in_specs=[pl.BlockSpec(memory_space=pl.ANY)],
        out_specs=pl.BlockSpec(memory_space=pltpu.MemorySpace.HBM),
        scratch_shapes=[
            pltpu.VMEM((QR, n), jnp.float32),
            pltpu.VMEM((QR, n), jnp.bfloat16),
            pltpu.VMEM((QR, n), jnp.bfloat16),
            pltpu.VMEM((QR, n), jnp.bfloat16),
            pltpu.VMEM((QR, n), jnp.bfloat16),
            pltpu.VMEM((QR, n), jnp.bfloat16),
            pltpu.VMEM((QR, n), jnp.bfloat16),
            pltpu.SemaphoreType.DMA,
            pltpu.SemaphoreType.DMA((NSEM,)),
            pltpu.SemaphoreType.DMA((NSEM,)),
            pltpu.SemaphoreType.DMA((NSEM,)),
        ],
        compiler_params=pltpu.CompilerParams(collective_id=0),
    )(x)


# device time: 50341 ns/iter; 1.0190x vs baseline; 1.0190x over previous
import jax
import jax.numpy as jnp
from jax import lax
from jax.experimental import pallas as pl
from jax.experimental.pallas import tpu as pltpu

QR = 1024
NSUB = 8
HALF = NSUB // 2
SUB = QR // NSUB

ZRAW = 0 * NSUB
XQ = 1 * NSUB
YQ = 2 * NSUB
FWDX = 3 * NSUB
FWDY = 3 * NSUB + HALF
NSEM = 4 * NSUB

O_ME, O_X, O_Y, O_D = 0, NSUB, 2 * NSUB, 3 * NSUB


def kernel(x):
    m, n = x.shape

    def body(
        x_hbm, out_hbm, lraw, sraw, praw, red, qx, qy, qd,
        local_sem, out_sems, send_sems, recv_sems,
    ):
        mx = lax.axis_index("x")
        my = lax.axis_index("y")
        mz = lax.axis_index("z")
        nbr_x = (1 - mx, my, mz)
        nbr_y = (mx, 1 - my, mz)
        nbr_z = (mx, my, 1 - mz)

        q_me = 2 * mx + my
        q_x = 2 * (1 - mx) + my
        q_y = 2 * mx + (1 - my)
        q_d = 2 * (1 - mx) + (1 - my)

        def sub(ref, s):
            return ref.at[pl.ds(s * SUB, SUB), :]

        def copy(slot, src, dst, target):
            return pltpu.make_async_remote_copy(
                src_ref=src,
                dst_ref=dst,
                send_sem=send_sems.at[slot],
                recv_sem=recv_sems.at[slot],
                device_id=target,
                device_id_type=pl.DeviceIdType.MESH,
            )

        out_dmas = []

        def out_sub(slot, src_ref, s, quarter):
            d = pltpu.make_async_copy(
                sub(src_ref, s),
                out_hbm.at[pl.ds(quarter * QR + s * SUB, SUB), :],
                out_sems.at[slot + s],
            )
            d.start()
            out_dmas.append(d)

        ldma = pltpu.make_async_copy(
            x_hbm.at[pl.ds(q_me * QR, QR), :], lraw, local_sem
        )
        ldma.start()
        ldma.wait()
        sraw[pl.ds(0, SUB), :] = lraw[pl.ds(0, SUB), :].astype(jnp.bfloat16)

        barrier_sem = pltpu.get_barrier_semaphore()
        for nbr in (nbr_x, nbr_y, nbr_z):
            pl.semaphore_signal(
                barrier_sem, inc=1,
                device_id=nbr, device_id_type=pl.DeviceIdType.MESH,
            )
        pl.semaphore_wait(barrier_sem, 3)

        sends = []

        rz = []
        for s in range(NSUB):
            if s:
                sraw[pl.ds(s * SUB, SUB), :] = lraw[
                    pl.ds(s * SUB, SUB), :
                ].astype(jnp.bfloat16)
            r = copy(ZRAW + s, sub(sraw, s), sub(praw, s), nbr_z)
            r.start()
            rz.append(r)
            sends.append(r)

        rqx, rqy = {}, {}
        for s in range(NSUB):
            rz[s].wait_recv()
            red_f32 = (
                lraw[pl.ds(s * SUB, SUB), :]
                + praw[pl.ds(s * SUB, SUB), :].astype(jnp.float32)
            )
            red[pl.ds(s * SUB, SUB), :] = red_f32.astype(jnp.bfloat16)
            rx = copy(XQ + s, sub(red, s), sub(qx, s), nbr_x)
            ry = copy(YQ + s, sub(red, s), sub(qy, s), nbr_y)
            rx.start()
            ry.start()
            rqx[s], rqy[s] = rx, ry
            sends.extend((rx, ry))
            out_sub(O_ME, red, s, q_me)

        for s in range(HALF):
            rqy[s].wait_recv()
            f = copy(FWDX + s, sub(qy, s), sub(qd, s), nbr_x)
            f.start()
            sends.append(f)
            out_sub(O_Y, qy, s, q_y)
        for s in range(HALF, NSUB):
            rqx[s].wait_recv()
            f = copy(FWDY + (s - HALF), sub(qx, s), sub(qd, s), nbr_y)
            f.start()
            sends.append(f)
            out_sub(O_X, qx, s, q_x)

        for s in range(HALF):
            rqx[s].wait_recv()
            out_sub(O_X, qx, s, q_x)
        for s in range(HALF, NSUB):
            rqy[s].wait_recv()
            out_sub(O_Y, qy, s, q_y)

        for s in range(NSUB):
            slot = FWDX + s if s < HALF else FWDY + (s - HALF)
            src_nbr = nbr_x if s < HALF else nbr_y
            copy(slot, sub(qy, s), sub(qd, s), src_nbr).wait_recv()
            out_sub(O_D, qd, s, q_d)

        for d in sends:
            d.wait_send()
        for d in out_dmas:
            d.wait()

    return pl.pallas_call(
        body,
        out_shape=jax.ShapeDtypeStruct((m, n), jnp.bfloat16),
        in_specs=[pl.BlockSpec(memory_space=pl.ANY)],
        out_specs=pl.BlockSpec(memory_space=pltpu.MemorySpace.HBM),
        scratch_shapes=[
            pltpu.VMEM((QR, n), jnp.float32),
            pltpu.VMEM((QR, n), jnp.bfloat16),
            pltpu.VMEM((QR, n), jnp.bfloat16),
            pltpu.VMEM((QR, n), jnp.bfloat16),
            pltpu.VMEM((QR, n), jnp.bfloat16),
            pltpu.VMEM((QR, n), jnp.bfloat16),
            pltpu.VMEM((QR, n), jnp.bfloat16),
            pltpu.SemaphoreType.DMA,
            pltpu.SemaphoreType.DMA((NSEM,)),
            pltpu.SemaphoreType.DMA((NSEM,)),
            pltpu.SemaphoreType.DMA((NSEM,)),
        ],
        compiler_params=pltpu.CompilerParams(collective_id=0),
    )(x)
